# baseline (speedup 1.0000x reference)
import functools

import jax
import jax.numpy as jnp
from jax import lax
from jax.experimental import pallas as pl
from jax.experimental.pallas import tpu as pltpu
from jax.experimental.pallas import tpu_sc as plsc

E = 320000
NUM_CATS = 9
D = 128
NC = 2
NS = 16
NW = NC * NS
B_PER_W = E // NW
BLK = 128
FULL_BLKS = B_PER_W // BLK
TAIL = B_PER_W - FULL_BLKS * BLK


def _lookup_body(idx_hbm, table_hbm, out_hbm, idx_v, rows_v, sem):
    c = lax.axis_index("c")
    s = lax.axis_index("s")
    wid = s * NC + c
    base = wid * B_PER_W
    pltpu.sync_copy(idx_hbm.at[pl.ds(base, B_PER_W)], idx_v)

    def body(j, carry):
        off = j * BLK
        pltpu.async_copy(
            table_hbm.at[idx_v.at[pl.ds(off, BLK)]], rows_v, sem
        ).wait()
        pltpu.sync_copy(rows_v, out_hbm.at[pl.ds(base + off, BLK)])
        return carry

    lax.fori_loop(0, FULL_BLKS, body, 0)

    toff = FULL_BLKS * BLK
    pltpu.async_copy(
        table_hbm.at[idx_v.at[pl.ds(toff, TAIL)]],
        rows_v.at[pl.ds(0, TAIL)],
        sem,
    ).wait()
    pltpu.sync_copy(rows_v.at[pl.ds(0, TAIL)], out_hbm.at[pl.ds(base + toff, TAIL)])


@functools.partial(
    pl.kernel,
    mesh=plsc.VectorSubcoreMesh(core_axis_name="c", subcore_axis_name="s"),
    out_type=jax.ShapeDtypeStruct((E, D), jnp.float32),
    scratch_types=[
        pltpu.VMEM((B_PER_W,), jnp.int32),
        pltpu.VMEM((BLK, D), jnp.float32),
        pltpu.SemaphoreType.DMA,
    ],
)
def _lookup(idx_hbm, table_hbm, out_hbm, idx_v, rows_v, sem):
    _lookup_body(idx_hbm, table_hbm, out_hbm, idx_v, rows_v, sem)


def kernel(edge_attr, table0):
    idx = edge_attr.reshape(E)
    return _lookup(idx, table0)


# speedup vs baseline: 1.0056x; 1.0056x over previous
import functools

import jax
import jax.numpy as jnp
from jax import lax
from jax.experimental import pallas as pl
from jax.experimental.pallas import tpu as pltpu
from jax.experimental.pallas import tpu_sc as plsc

E = 320000
NUM_CATS = 9
D = 128
NC = 2
NS = 16
NW = NC * NS
B_PER_W = E // NW
BLK = 128
NBUF = 6
FULL_BLKS = B_PER_W // BLK
NGROUPS = FULL_BLKS // NBUF
TAIL = B_PER_W - FULL_BLKS * BLK


def _lookup_body(idx_hbm, table_hbm, out_hbm, idx_v, rows_v, gsems, wsems):
    c = lax.axis_index("c")
    s = lax.axis_index("s")
    wid = s * NC + c
    base = wid * B_PER_W
    pltpu.sync_copy(idx_hbm.at[pl.ds(base, B_PER_W)], idx_v)

    def start_gather(j, b):
        pltpu.make_async_copy(
            table_hbm.at[idx_v.at[pl.ds(j * BLK, BLK)]], rows_v.at[b], gsems[b]
        ).start()

    def wait_gather(b):
        pltpu.make_async_copy(
            table_hbm.at[idx_v.at[pl.ds(0, BLK)]], rows_v.at[b], gsems[b]
        ).wait()

    def start_write(j, b):
        pltpu.make_async_copy(
            rows_v.at[b], out_hbm.at[pl.ds(base + j * BLK, BLK)], wsems[b]
        ).start()

    def wait_write(b):
        pltpu.make_async_copy(
            rows_v.at[b], out_hbm.at[pl.ds(base, BLK)], wsems[b]
        ).wait()

    for b in range(NBUF):
        start_gather(b, b)

    def group(o, carry):
        for b in range(NBUF):
            wait_gather(b)
            start_write(o * NBUF + b, b)

        @pl.when(o + 1 < NGROUPS)
        def _():
            for b in range(NBUF):
                wait_write(b)
                start_gather((o + 1) * NBUF + b, b)

        return carry

    lax.fori_loop(0, NGROUPS, group, 0)

    for b in range(NBUF):
        wait_write(b)

    toff = FULL_BLKS * BLK
    pltpu.async_copy(
        table_hbm.at[idx_v.at[pl.ds(toff, TAIL)]],
        rows_v.at[0, pl.ds(0, TAIL)],
        gsems[0],
    ).wait()
    pltpu.sync_copy(
        rows_v.at[0, pl.ds(0, TAIL)], out_hbm.at[pl.ds(base + toff, TAIL)]
    )


@functools.partial(
    pl.kernel,
    mesh=plsc.VectorSubcoreMesh(core_axis_name="c", subcore_axis_name="s"),
    out_type=jax.ShapeDtypeStruct((E, D), jnp.float32),
    scratch_types=[
        pltpu.VMEM((B_PER_W,), jnp.int32),
        pltpu.VMEM((NBUF, BLK, D), jnp.float32),
    ]
    + [pltpu.SemaphoreType.DMA] * (2 * NBUF),
)
def _lookup(idx_hbm, table_hbm, out_hbm, idx_v, rows_v, *sems):
    _lookup_body(
        idx_hbm, table_hbm, out_hbm, idx_v, rows_v, sems[:NBUF], sems[NBUF:]
    )


def kernel(edge_attr, table0):
    idx = edge_attr.reshape(E)
    return _lookup(idx, table0)


# speedup vs baseline: 4.0903x; 4.0674x over previous
import functools

import jax
import jax.numpy as jnp
from jax import lax
from jax.experimental import pallas as pl
from jax.experimental.pallas import tpu as pltpu
from jax.experimental.pallas import tpu_sc as plsc

E = 320000
NUM_CATS = 9
D = 128
NC = 2
NS = 16
NW = NC * NS
B_PER_W = E // NW
REP = 16
BLK = 128
NBUF = 6
FULL_BLKS = B_PER_W // BLK
NGROUPS = FULL_BLKS // NBUF
TAIL = B_PER_W - FULL_BLKS * BLK


def _lookup_body(idx_hbm, table_hbm, out_hbm, idx_v, rows_v, gsems, wsems):
    c = lax.axis_index("c")
    s = lax.axis_index("s")
    wid = s * NC + c
    base = wid * B_PER_W
    pltpu.sync_copy(idx_hbm.at[pl.ds(base, B_PER_W)], idx_v)

    lane = lax.iota(jnp.int32, 16)

    def retarget(i, carry):
        sl = pl.ds(i * 16, 16)
        idx_v[sl] = idx_v[sl] * REP + lane
        return carry

    lax.fori_loop(0, B_PER_W // 16, retarget, 0)

    def start_gather(j, b):
        pltpu.make_async_copy(
            table_hbm.at[idx_v.at[pl.ds(j * BLK, BLK)]], rows_v.at[b], gsems[b]
        ).start()

    def wait_gather(b):
        pltpu.make_async_copy(
            table_hbm.at[idx_v.at[pl.ds(0, BLK)]], rows_v.at[b], gsems[b]
        ).wait()

    def start_write(j, b):
        pltpu.make_async_copy(
            rows_v.at[b], out_hbm.at[pl.ds(base + j * BLK, BLK)], wsems[b]
        ).start()

    def wait_write(b):
        pltpu.make_async_copy(
            rows_v.at[b], out_hbm.at[pl.ds(base, BLK)], wsems[b]
        ).wait()

    for b in range(NBUF):
        start_gather(b, b)

    def group(o, carry):
        for b in range(NBUF):
            wait_gather(b)
            start_write(o * NBUF + b, b)

        @pl.when(o + 1 < NGROUPS)
        def _():
            for b in range(NBUF):
                wait_write(b)
                start_gather((o + 1) * NBUF + b, b)

        return carry

    lax.fori_loop(0, NGROUPS, group, 0)

    for b in range(NBUF):
        wait_write(b)

    toff = FULL_BLKS * BLK
    pltpu.async_copy(
        table_hbm.at[idx_v.at[pl.ds(toff, TAIL)]],
        rows_v.at[0, pl.ds(0, TAIL)],
        gsems[0],
    ).wait()
    pltpu.sync_copy(
        rows_v.at[0, pl.ds(0, TAIL)], out_hbm.at[pl.ds(base + toff, TAIL)]
    )


@functools.partial(
    pl.kernel,
    mesh=plsc.VectorSubcoreMesh(core_axis_name="c", subcore_axis_name="s"),
    out_type=jax.ShapeDtypeStruct((E, D), jnp.float32),
    scratch_types=[
        pltpu.VMEM((B_PER_W,), jnp.int32),
        pltpu.VMEM((NBUF, BLK, D), jnp.float32),
    ]
    + [pltpu.SemaphoreType.DMA] * (2 * NBUF),
)
def _lookup(idx_hbm, table_hbm, out_hbm, idx_v, rows_v, *sems):
    _lookup_body(
        idx_hbm, table_hbm, out_hbm, idx_v, rows_v, sems[:NBUF], sems[NBUF:]
    )


def kernel(edge_attr, table0):
    idx = edge_attr.reshape(E)
    table_rep = jnp.repeat(table0, REP, axis=0)
    return _lookup(idx, table_rep)


# speedup vs baseline: 16.9476x; 4.1434x over previous
import functools

import jax
import jax.numpy as jnp
from jax import lax
from jax.experimental import pallas as pl
from jax.experimental.pallas import tpu as pltpu
from jax.experimental.pallas import tpu_sc as plsc

E = 320000
NUM_CATS = 9
D = 128
NC = 2
NS = 16
NW = NC * NS
B_PER_W = E // NW
BLK = 128
NBUF = 6
FULL_BLKS = B_PER_W // BLK
NGROUPS = FULL_BLKS // NBUF
TAIL = B_PER_W - FULL_BLKS * BLK


def _lookup_body(idx_hbm, table_hbm, out_hbm, idx_v, rows_v, table_sh, gsems, wsems):
    c = lax.axis_index("c")
    s = lax.axis_index("s")
    wid = s * NC + c
    base = wid * B_PER_W

    pltpu.sync_copy(table_hbm, table_sh.at[pl.ds(s * NUM_CATS, NUM_CATS)])
    pltpu.sync_copy(idx_hbm.at[pl.ds(base, B_PER_W)], idx_v)

    roff = s * NUM_CATS

    def retarget(i, carry):
        sl = pl.ds(i * 16, 16)
        idx_v[sl] = idx_v[sl] + roff
        return carry

    lax.fori_loop(0, B_PER_W // 16, retarget, 0)

    def start_gather(j, b):
        pltpu.make_async_copy(
            table_sh.at[idx_v.at[pl.ds(j * BLK, BLK)]], rows_v.at[b], gsems[b]
        ).start()

    def wait_gather(b):
        pltpu.make_async_copy(
            table_sh.at[idx_v.at[pl.ds(0, BLK)]], rows_v.at[b], gsems[b]
        ).wait()

    def start_write(j, b):
        pltpu.make_async_copy(
            rows_v.at[b], out_hbm.at[pl.ds(base + j * BLK, BLK)], wsems[b]
        ).start()

    def wait_write(b):
        pltpu.make_async_copy(
            rows_v.at[b], out_hbm.at[pl.ds(base, BLK)], wsems[b]
        ).wait()

    for b in range(NBUF):
        start_gather(b, b)

    def group(o, carry):
        for b in range(NBUF):
            wait_gather(b)
            start_write(o * NBUF + b, b)

        @pl.when(o + 1 < NGROUPS)
        def _():
            for b in range(NBUF):
                wait_write(b)
                start_gather((o + 1) * NBUF + b, b)

        return carry

    lax.fori_loop(0, NGROUPS, group, 0)

    for b in range(NBUF):
        wait_write(b)

    toff = FULL_BLKS * BLK
    pltpu.async_copy(
        table_sh.at[idx_v.at[pl.ds(toff, TAIL)]],
        rows_v.at[0, pl.ds(0, TAIL)],
        gsems[0],
    ).wait()
    pltpu.sync_copy(
        rows_v.at[0, pl.ds(0, TAIL)], out_hbm.at[pl.ds(base + toff, TAIL)]
    )


@functools.partial(
    pl.kernel,
    mesh=plsc.VectorSubcoreMesh(core_axis_name="c", subcore_axis_name="s"),
    out_type=jax.ShapeDtypeStruct((E, D), jnp.float32),
    scratch_types=[
        pltpu.VMEM((B_PER_W,), jnp.int32),
        pltpu.VMEM((NBUF, BLK, D), jnp.float32),
        pltpu.VMEM_SHARED((NS * NUM_CATS, D), jnp.float32),
    ]
    + [pltpu.SemaphoreType.DMA] * (2 * NBUF),
)
def _lookup(idx_hbm, table_hbm, out_hbm, idx_v, rows_v, table_sh, *sems):
    _lookup_body(
        idx_hbm, table_hbm, out_hbm, idx_v, rows_v, table_sh,
        sems[:NBUF], sems[NBUF:],
    )


def kernel(edge_attr, table0):
    idx = edge_attr.reshape(E)
    return _lookup(idx, table0)


# speedup vs baseline: 17.3970x; 1.0265x over previous
import functools

import jax
import jax.numpy as jnp
from jax import lax
from jax.experimental import pallas as pl
from jax.experimental.pallas import tpu as pltpu
from jax.experimental.pallas import tpu_sc as plsc

E = 320000
NUM_CATS = 9
D = 128
NC = 2
NS = 16
NW = NC * NS
B_PER_W = E // NW
BLK = 128
NBUF = 6
FULL_BLKS = B_PER_W // BLK
NGROUPS = FULL_BLKS // NBUF
TAIL = B_PER_W - FULL_BLKS * BLK


def _lookup_body(idx_hbm, table_hbm, out_hbm, idx_v, rows_v, table_sh, gsems, wsems):
    c = lax.axis_index("c")
    s = lax.axis_index("s")
    wid = s * NC + c
    base = wid * B_PER_W

    pltpu.sync_copy(table_hbm, table_sh.at[pl.ds(s * NUM_CATS, NUM_CATS)])
    pltpu.sync_copy(idx_hbm.at[pl.ds(base, B_PER_W)], idx_v)

    roff = s * NUM_CATS

    def retarget(j, nvec):
        def step(i, carry):
            sl = pl.ds(j * BLK + i * 16, 16)
            idx_v[sl] = idx_v[sl] + roff
            return carry

        lax.fori_loop(0, nvec, step, 0)

    def start_gather(j, b):
        pltpu.make_async_copy(
            table_sh.at[idx_v.at[pl.ds(j * BLK, BLK)]], rows_v.at[b], gsems[b]
        ).start()

    def wait_gather(b):
        pltpu.make_async_copy(
            table_sh.at[idx_v.at[pl.ds(0, BLK)]], rows_v.at[b], gsems[b]
        ).wait()

    def start_write(j, b):
        pltpu.make_async_copy(
            rows_v.at[b], out_hbm.at[pl.ds(base + j * BLK, BLK)], wsems[b]
        ).start()

    def wait_write(b):
        pltpu.make_async_copy(
            rows_v.at[b], out_hbm.at[pl.ds(base, BLK)], wsems[b]
        ).wait()

    for b in range(NBUF):
        retarget(b, BLK // 16)
        start_gather(b, b)

    def group(o, carry):
        @pl.when(o + 1 < NGROUPS)
        def _():
            for b in range(NBUF):
                retarget((o + 1) * NBUF + b, BLK // 16)

        for b in range(NBUF):
            wait_gather(b)
            start_write(o * NBUF + b, b)

        @pl.when(o + 1 < NGROUPS)
        def _():
            for b in range(NBUF):
                wait_write(b)
                start_gather((o + 1) * NBUF + b, b)

        return carry

    lax.fori_loop(0, NGROUPS, group, 0)

    for b in range(NBUF):
        wait_write(b)

    toff = FULL_BLKS * BLK
    retarget(FULL_BLKS, TAIL // 16)
    pltpu.async_copy(
        table_sh.at[idx_v.at[pl.ds(toff, TAIL)]],
        rows_v.at[0, pl.ds(0, TAIL)],
        gsems[0],
    ).wait()
    pltpu.sync_copy(
        rows_v.at[0, pl.ds(0, TAIL)], out_hbm.at[pl.ds(base + toff, TAIL)]
    )


@functools.partial(
    pl.kernel,
    mesh=plsc.VectorSubcoreMesh(core_axis_name="c", subcore_axis_name="s"),
    out_type=jax.ShapeDtypeStruct((E, D), jnp.float32),
    scratch_types=[
        pltpu.VMEM((B_PER_W,), jnp.int32),
        pltpu.VMEM((NBUF, BLK, D), jnp.float32),
        pltpu.VMEM_SHARED((NS * NUM_CATS, D), jnp.float32),
    ]
    + [pltpu.SemaphoreType.DMA] * (2 * NBUF),
)
def _lookup(idx_hbm, table_hbm, out_hbm, idx_v, rows_v, table_sh, *sems):
    _lookup_body(
        idx_hbm, table_hbm, out_hbm, idx_v, rows_v, table_sh,
        sems[:NBUF], sems[NBUF:],
    )


def kernel(edge_attr, table0):
    idx = edge_attr.reshape(E)
    return _lookup(idx, table0)
